# baseline (device time: 99318 ns/iter reference)
import jax
import jax.numpy as jnp
from jax import lax
from jax.experimental import pallas as pl
from jax.experimental.pallas import tpu as pltpu

N_DEV = 4
SUB = 4


def kernel(x, w_mat):
    m, k_per = x.shape
    _, n = w_mat.shape
    m_per = m // N_DEV
    n_half = n // 2

    def body(x_hbm, w_ref, out_ref,
             xbuf, stageA, stageB, commA, commB, amax_ref,
             xsems, sendA, recvA, sendB, recvB, amax_send_sems,
             amax_recv_sems):
        my = lax.axis_index("i")
        left = lax.rem(my + N_DEV - 1, N_DEV)
        right = lax.rem(my + 1, N_DEV)

        def xcopy(c, slot):
            return pltpu.make_async_copy(
                x_hbm.at[pl.ds(c * m_per, m_per), :],
                xbuf.at[slot], xsems.at[slot])

        def ring_rdma(src, dst, ssem, rsem, target):
            return pltpu.make_async_remote_copy(
                src_ref=src, dst_ref=dst, send_sem=ssem, recv_sem=rsem,
                device_id=(target,), device_id_type=pl.DeviceIdType.MESH)

        xcopy(right, 0).start()
        xcopy(lax.rem(my + 3, N_DEV), 1).start()

        barrier = pltpu.get_barrier_semaphore()
        for off in (1, 2, 3):
            pl.semaphore_signal(barrier, inc=1,
                                device_id=(lax.rem(my + off, N_DEV),),
                                device_id_type=pl.DeviceIdType.MESH)
        pl.semaphore_wait(barrier, N_DEV - 1)

        wA = w_ref[:, pl.ds(0, n_half)].astype(jnp.bfloat16)
        wB = w_ref[:, pl.ds(n_half, n_half)].astype(jnp.bfloat16)

        drain = []
        sub_m = m_per // SUB

        def rsub(q):
            return pl.ds(q * sub_m, sub_m)

        def ring_send(src_ref, dst_ref, ssems, rsems, s, q, target):
            rdma = ring_rdma(src_ref.at[rsub(q)] if s == 0
                             else src_ref.at[s - 1, rsub(q)],
                             dst_ref.at[s, rsub(q)],
                             ssems.at[s, q], rsems.at[s, q], target)
            rdma.start()
            drain.append(rdma)
            return rdma

        xcopy(right, 0).wait()
        x1 = xbuf[0].astype(jnp.bfloat16)
        xcopy(lax.rem(my + 2, N_DEV), 0).start()
        stageA[...] = jnp.dot(
            x1, wA, preferred_element_type=jnp.float32).astype(jnp.bfloat16)
        a0 = [ring_send(stageA, commA, sendA, recvA, 0, q, left)
              for q in range(SUB)]

        xcopy(lax.rem(my + 3, N_DEV), 1).wait()
        x3 = xbuf[1].astype(jnp.bfloat16)
        xcopy(my, 1).start()
        stageB[...] = jnp.dot(
            x3, wB, preferred_element_type=jnp.float32).astype(jnp.bfloat16)
        b0 = [ring_send(stageB, commB, sendB, recvB, 0, q, right)
              for q in range(SUB)]

        xcopy(lax.rem(my + 2, N_DEV), 0).wait()
        x2 = xbuf[0].astype(jnp.bfloat16)
        pA1 = jnp.dot(x2, wA, preferred_element_type=jnp.float32)
        pB1 = jnp.dot(x2, wB, preferred_element_type=jnp.float32)

        a1, b1 = [], []
        for q in range(SUB):
            a0[q].wait_recv()
            commA[0, rsub(q)] = (commA[0, rsub(q)].astype(jnp.float32)
                                 + pA1[q * sub_m:(q + 1) * sub_m]
                                 ).astype(jnp.bfloat16)
            a1.append(ring_send(commA, commA, sendA, recvA, 1, q, left))
            b0[q].wait_recv()
            commB[0, rsub(q)] = (commB[0, rsub(q)].astype(jnp.float32)
                                 + pB1[q * sub_m:(q + 1) * sub_m]
                                 ).astype(jnp.bfloat16)
            b1.append(ring_send(commB, commB, sendB, recvB, 1, q, right))

        pA2 = jnp.dot(x3, wA, preferred_element_type=jnp.float32)
        pB2 = jnp.dot(x1, wB, preferred_element_type=jnp.float32)

        a2, b2 = [], []
        for q in range(SUB):
            a1[q].wait_recv()
            commA[1, rsub(q)] = (commA[1, rsub(q)].astype(jnp.float32)
                                 + pA2[q * sub_m:(q + 1) * sub_m]
                                 ).astype(jnp.bfloat16)
            a2.append(ring_send(commA, commA, sendA, recvA, 2, q, left))
            b1[q].wait_recv()
            commB[1, rsub(q)] = (commB[1, rsub(q)].astype(jnp.float32)
                                 + pB2[q * sub_m:(q + 1) * sub_m]
                                 ).astype(jnp.bfloat16)
            b2.append(ring_send(commB, commB, sendB, recvB, 2, q, right))

        xcopy(my, 1).wait()
        x0 = xbuf[1].astype(jnp.bfloat16)
        pownA = jnp.dot(x0, wA, preferred_element_type=jnp.float32)
        pownB = jnp.dot(x0, wB, preferred_element_type=jnp.float32)

        for q in range(SUB):
            a2[q].wait_recv()
            out_ref[rsub(q), pl.ds(0, n_half)] = (
                commA[2, rsub(q)].astype(jnp.float32)
                + pownA[q * sub_m:(q + 1) * sub_m])
            b2[q].wait_recv()
            out_ref[rsub(q), pl.ds(n_half, n_half)] = (
                commB[2, rsub(q)].astype(jnp.float32)
                + pownB[q * sub_m:(q + 1) * sub_m])

        local_amax = jnp.maximum(jnp.max(out_ref[...]), 0.0)
        amax_ref[3] = jnp.full((8, 128), local_amax, dtype=jnp.float32)
        amax_descs = []
        for j in range(N_DEV - 1):
            dest = lax.rem(my + 1 + j, N_DEV)
            rdma = pltpu.make_async_remote_copy(
                src_ref=amax_ref.at[3],
                dst_ref=amax_ref.at[j],
                send_sem=amax_send_sems.at[j],
                recv_sem=amax_recv_sems.at[j],
                device_id=(dest,),
                device_id_type=pl.DeviceIdType.MESH,
            )
            rdma.start()
            amax_descs.append(rdma)
        for rdma in amax_descs:
            rdma.wait_recv()
            rdma.wait_send()

        gmax = jnp.max(amax_ref[...])
        scale = gmax * (1.0 / 448.0)
        inv = 448.0 / gmax
        for h in range(2):
            sl = pl.ds(h * n_half, n_half)
            q = (jnp.maximum(out_ref[:, sl], 0.0) * inv).astype(
                jnp.float8_e4m3fn)
            out_ref[:, sl] = q.astype(jnp.float32) * scale

        for rdma in drain:
            rdma.wait_send()

    return pl.pallas_call(
        body,
        out_shape=jax.ShapeDtypeStruct((m_per, n), jnp.float32),
        in_specs=[pl.BlockSpec(memory_space=pl.ANY),
                  pl.BlockSpec(memory_space=pltpu.VMEM)],
        out_specs=pl.BlockSpec(memory_space=pltpu.VMEM),
        scratch_shapes=[
            pltpu.VMEM((2, m_per, k_per), jnp.float32),
            pltpu.VMEM((m_per, n_half), jnp.bfloat16),
            pltpu.VMEM((m_per, n_half), jnp.bfloat16),
            pltpu.VMEM((3, m_per, n_half), jnp.bfloat16),
            pltpu.VMEM((3, m_per, n_half), jnp.bfloat16),
            pltpu.VMEM((N_DEV, 8, 128), jnp.float32),
            pltpu.SemaphoreType.DMA((2,)),
            pltpu.SemaphoreType.DMA((3, SUB)),
            pltpu.SemaphoreType.DMA((3, SUB)),
            pltpu.SemaphoreType.DMA((3, SUB)),
            pltpu.SemaphoreType.DMA((3, SUB)),
            pltpu.SemaphoreType.DMA((N_DEV - 1,)),
            pltpu.SemaphoreType.DMA((N_DEV - 1,)),
        ],
        compiler_params=pltpu.CompilerParams(
            collective_id=0, vmem_limit_bytes=100 * 1024 * 1024),
    )(x, w_mat)


# device time: 96577 ns/iter; 1.0284x vs baseline; 1.0284x over previous
import jax
import jax.numpy as jnp
from jax import lax
from jax.experimental import pallas as pl
from jax.experimental.pallas import tpu as pltpu

N_DEV = 4
SUB = 2


def kernel(x, w_mat):
    m, k_per = x.shape
    _, n = w_mat.shape
    m_per = m // N_DEV
    n_half = n // 2

    def body(x_hbm, w_ref, out_ref,
             xbuf, stageA, stageB, commA, commB, amax_ref,
             xsems, sendA, recvA, sendB, recvB, amax_send_sems,
             amax_recv_sems):
        my = lax.axis_index("i")
        left = lax.rem(my + N_DEV - 1, N_DEV)
        right = lax.rem(my + 1, N_DEV)

        def xcopy(c, slot):
            return pltpu.make_async_copy(
                x_hbm.at[pl.ds(c * m_per, m_per), :],
                xbuf.at[slot], xsems.at[slot])

        def ring_rdma(src, dst, ssem, rsem, target):
            return pltpu.make_async_remote_copy(
                src_ref=src, dst_ref=dst, send_sem=ssem, recv_sem=rsem,
                device_id=(target,), device_id_type=pl.DeviceIdType.MESH)

        xcopy(right, 0).start()
        xcopy(lax.rem(my + 3, N_DEV), 1).start()

        barrier = pltpu.get_barrier_semaphore()
        for off in (1, 2, 3):
            pl.semaphore_signal(barrier, inc=1,
                                device_id=(lax.rem(my + off, N_DEV),),
                                device_id_type=pl.DeviceIdType.MESH)
        pl.semaphore_wait(barrier, N_DEV - 1)

        wA = w_ref[:, pl.ds(0, n_half)].astype(jnp.bfloat16)
        wB = w_ref[:, pl.ds(n_half, n_half)].astype(jnp.bfloat16)

        drain = []
        sub_m = m_per // SUB

        def rsub(q):
            return pl.ds(q * sub_m, sub_m)

        def ring_send(src_ref, dst_ref, ssems, rsems, s, q, target):
            rdma = ring_rdma(src_ref.at[rsub(q)] if s == 0
                             else src_ref.at[s - 1, rsub(q)],
                             dst_ref.at[s, rsub(q)],
                             ssems.at[s, q], rsems.at[s, q], target)
            rdma.start()
            drain.append(rdma)
            return rdma

        xcopy(right, 0).wait()
        x1 = xbuf[0].astype(jnp.bfloat16)
        xcopy(lax.rem(my + 2, N_DEV), 0).start()
        a0 = []
        for q in range(SUB):
            stageA[rsub(q)] = jnp.dot(
                x1[q * sub_m:(q + 1) * sub_m], wA,
                preferred_element_type=jnp.float32).astype(jnp.bfloat16)
            a0.append(ring_send(stageA, commA, sendA, recvA, 0, q, left))

        xcopy(lax.rem(my + 3, N_DEV), 1).wait()
        x3 = xbuf[1].astype(jnp.bfloat16)
        xcopy(my, 1).start()
        b0 = []
        for q in range(SUB):
            stageB[rsub(q)] = jnp.dot(
                x3[q * sub_m:(q + 1) * sub_m], wB,
                preferred_element_type=jnp.float32).astype(jnp.bfloat16)
            b0.append(ring_send(stageB, commB, sendB, recvB, 0, q, right))

        xcopy(lax.rem(my + 2, N_DEV), 0).wait()
        x2 = xbuf[0].astype(jnp.bfloat16)
        pA1 = jnp.dot(x2, wA, preferred_element_type=jnp.float32)
        pB1 = jnp.dot(x2, wB, preferred_element_type=jnp.float32)

        a1, b1 = [], []
        for q in range(SUB):
            a0[q].wait_recv()
            commA[0, rsub(q)] = (commA[0, rsub(q)].astype(jnp.float32)
                                 + pA1[q * sub_m:(q + 1) * sub_m]
                                 ).astype(jnp.bfloat16)
            a1.append(ring_send(commA, commA, sendA, recvA, 1, q, left))
            b0[q].wait_recv()
            commB[0, rsub(q)] = (commB[0, rsub(q)].astype(jnp.float32)
                                 + pB1[q * sub_m:(q + 1) * sub_m]
                                 ).astype(jnp.bfloat16)
            b1.append(ring_send(commB, commB, sendB, recvB, 1, q, right))

        pA2 = jnp.dot(x3, wA, preferred_element_type=jnp.float32)
        pB2 = jnp.dot(x1, wB, preferred_element_type=jnp.float32)

        a2, b2 = [], []
        for q in range(SUB):
            a1[q].wait_recv()
            commA[1, rsub(q)] = (commA[1, rsub(q)].astype(jnp.float32)
                                 + pA2[q * sub_m:(q + 1) * sub_m]
                                 ).astype(jnp.bfloat16)
            a2.append(ring_send(commA, commA, sendA, recvA, 2, q, left))
            b1[q].wait_recv()
            commB[1, rsub(q)] = (commB[1, rsub(q)].astype(jnp.float32)
                                 + pB2[q * sub_m:(q + 1) * sub_m]
                                 ).astype(jnp.bfloat16)
            b2.append(ring_send(commB, commB, sendB, recvB, 2, q, right))

        xcopy(my, 1).wait()
        x0 = xbuf[1].astype(jnp.bfloat16)
        pownA = jnp.dot(x0, wA, preferred_element_type=jnp.float32)
        pownB = jnp.dot(x0, wB, preferred_element_type=jnp.float32)

        local_amax = jnp.float32(0.0)
        for q in range(SUB):
            a2[q].wait_recv()
            yA = (commA[2, rsub(q)].astype(jnp.float32)
                  + pownA[q * sub_m:(q + 1) * sub_m])
            out_ref[rsub(q), pl.ds(0, n_half)] = yA
            local_amax = jnp.maximum(local_amax, jnp.max(yA))
            b2[q].wait_recv()
            yB = (commB[2, rsub(q)].astype(jnp.float32)
                  + pownB[q * sub_m:(q + 1) * sub_m])
            out_ref[rsub(q), pl.ds(n_half, n_half)] = yB
            local_amax = jnp.maximum(local_amax, jnp.max(yB))

        amax_ref[3] = jnp.full((8, 128), local_amax, dtype=jnp.float32)
        amax_descs = []
        for j in range(N_DEV - 1):
            dest = lax.rem(my + 1 + j, N_DEV)
            rdma = pltpu.make_async_remote_copy(
                src_ref=amax_ref.at[3],
                dst_ref=amax_ref.at[j],
                send_sem=amax_send_sems.at[j],
                recv_sem=amax_recv_sems.at[j],
                device_id=(dest,),
                device_id_type=pl.DeviceIdType.MESH,
            )
            rdma.start()
            amax_descs.append(rdma)
        for rdma in amax_descs:
            rdma.wait_recv()
            rdma.wait_send()

        gmax = jnp.max(amax_ref[...])
        scale = gmax * (1.0 / 448.0)
        inv = 448.0 / gmax
        for h in range(2):
            sl = pl.ds(h * n_half, n_half)
            q = (jnp.maximum(out_ref[:, sl], 0.0) * inv).astype(
                jnp.float8_e4m3fn)
            out_ref[:, sl] = q.astype(jnp.float32) * scale

        for rdma in drain:
            rdma.wait_send()

    return pl.pallas_call(
        body,
        out_shape=jax.ShapeDtypeStruct((m_per, n), jnp.float32),
        in_specs=[pl.BlockSpec(memory_space=pl.ANY),
                  pl.BlockSpec(memory_space=pltpu.VMEM)],
        out_specs=pl.BlockSpec(memory_space=pltpu.VMEM),
        scratch_shapes=[
            pltpu.VMEM((2, m_per, k_per), jnp.float32),
            pltpu.VMEM((m_per, n_half), jnp.bfloat16),
            pltpu.VMEM((m_per, n_half), jnp.bfloat16),
            pltpu.VMEM((3, m_per, n_half), jnp.bfloat16),
            pltpu.VMEM((3, m_per, n_half), jnp.bfloat16),
            pltpu.VMEM((N_DEV, 8, 128), jnp.float32),
            pltpu.SemaphoreType.DMA((2,)),
            pltpu.SemaphoreType.DMA((3, SUB)),
            pltpu.SemaphoreType.DMA((3, SUB)),
            pltpu.SemaphoreType.DMA((3, SUB)),
            pltpu.SemaphoreType.DMA((3, SUB)),
            pltpu.SemaphoreType.DMA((N_DEV - 1,)),
            pltpu.SemaphoreType.DMA((N_DEV - 1,)),
        ],
        compiler_params=pltpu.CompilerParams(
            collective_id=0, vmem_limit_bytes=100 * 1024 * 1024),
    )(x, w_mat)
